# Initial kernel scaffold; baseline (speedup 1.0000x reference)
import jax
import jax.numpy as jnp
from jax.experimental import pallas as pl


def kernel(numerical_features, cat_features, tables, W1, b1, W2, b2, W3, b3, W4, b4):
    raise NotImplementedError("write your pallas kernel here")


# speedup vs baseline: 7.7294x; 7.7294x over previous
import functools

import jax
import jax.numpy as jnp
from jax.experimental import pallas as pl
from jax.experimental.pallas import tpu as pltpu
from jax.experimental.pallas import tpu_sc as plsc

VOCAB = 100000
EDIM = 32
NTAB = 26
B = 16384
NUM_DENSE = 13
CAT_DIM = NTAB * EDIM

GATHER_WINDOW = 128
BM = 2048


def _sc_gather(tab_flat, idx_flat):
    mesh = plsc.VectorSubcoreMesh(core_axis_name="core", subcore_axis_name="subcore")
    nrows = B * NTAB

    @functools.partial(
        pl.kernel,
        out_type=jax.ShapeDtypeStruct((nrows, EDIM), jnp.float32),
        mesh=mesh,
        compiler_params=pltpu.CompilerParams(use_tc_tiling_on_sc=False),
    )
    def gather_kernel(tab_hbm, idx_hbm, out_hbm):
        def body(idx_vmem, out_vmem):
            pltpu.sync_copy(tab_hbm.at[idx_vmem.at[0]], out_vmem)

        pltpu.emit_pipeline(
            body,
            grid=(nrows // GATHER_WINDOW,),
            in_specs=[
                pl.BlockSpec((1, GATHER_WINDOW), index_map=lambda i: (0, i))
            ],
            out_specs=[
                pl.BlockSpec((GATHER_WINDOW, EDIM), index_map=lambda i: (i, 0))
            ],
            core_axis_name=("core", "subcore"),
            dimension_semantics=(pltpu.PARALLEL,),
        )(idx_hbm, out_hbm)

    return gather_kernel(tab_flat, idx_flat)


def _mlp_body(cat_ref, num_ref, w1c_ref, w1n_ref, b1_ref, w2_ref, b2_ref,
              w3_ref, b3_ref, w4_ref, b4_ref, out_ref):
    h = jnp.dot(cat_ref[...], w1c_ref[...], preferred_element_type=jnp.float32)
    h = h + jnp.dot(num_ref[...], w1n_ref[...], preferred_element_type=jnp.float32)
    h = jnp.maximum(h + b1_ref[...], 0.0)
    h = jnp.maximum(
        jnp.dot(h, w2_ref[...], preferred_element_type=jnp.float32) + b2_ref[...], 0.0)
    h = jnp.maximum(
        jnp.dot(h, w3_ref[...], preferred_element_type=jnp.float32) + b3_ref[...], 0.0)
    out_ref[...] = jnp.dot(h, w4_ref[...], preferred_element_type=jnp.float32) + b4_ref[...]


def _mlp(cat_out, num, w1c, w1n, b1, w2, b2, w3, b3, w4, b4):
    nblk = B // BM
    full = lambda shape: pl.BlockSpec(shape, lambda i: (0, 0))
    return pl.pallas_call(
        _mlp_body,
        grid=(nblk,),
        in_specs=[
            pl.BlockSpec((BM, CAT_DIM), lambda i: (i, 0)),
            pl.BlockSpec((BM, NUM_DENSE), lambda i: (i, 0)),
            full((CAT_DIM, 512)),
            full((NUM_DENSE, 512)),
            full((1, 512)),
            full((512, 256)),
            full((1, 256)),
            full((256, 128)),
            full((1, 128)),
            full((128, 1)),
            full((1, 1)),
        ],
        out_specs=pl.BlockSpec((BM, 1), lambda i: (i, 0)),
        out_shape=jax.ShapeDtypeStruct((B, 1), jnp.float32),
    )(cat_out, num, w1c, w1n, b1, w2, b2, w3, b3, w4, b4)


def kernel(numerical_features, cat_features, tables, W1, b1, W2, b2, W3, b3, W4, b4):
    offsets = jnp.arange(NTAB, dtype=jnp.int32)[:, None] * VOCAB
    idx = jnp.mod(cat_features[:, :, 0], VOCAB) + offsets
    idx_flat = idx.T.reshape(1, B * NTAB)
    tab_flat = tables.reshape(NTAB * VOCAB, EDIM)
    cat_out = _sc_gather(tab_flat, idx_flat).reshape(B, CAT_DIM)
    out = _mlp(
        cat_out,
        numerical_features,
        W1[:CAT_DIM],
        W1[CAT_DIM:],
        b1.reshape(1, -1),
        W2,
        b2.reshape(1, -1),
        W3,
        b3.reshape(1, -1),
        W4,
        b4.reshape(1, -1),
    )
    return out


# speedup vs baseline: 25.3843x; 3.2841x over previous
import functools

import jax
import jax.numpy as jnp
from jax import lax
from jax.experimental import pallas as pl
from jax.experimental.pallas import tpu as pltpu
from jax.experimental.pallas import tpu_sc as plsc

VOCAB = 100000
EDIM = 32
NTAB = 26
B = 16384
NUM_DENSE = 13
CAT_DIM = NTAB * EDIM

BH = B // 2
BN = 2048


def _sc_gather_t(tabT, idx):
    mesh = plsc.VectorSubcoreMesh(core_axis_name="core", subcore_axis_name="subcore")

    @functools.partial(
        pl.kernel,
        out_type=jax.ShapeDtypeStruct((CAT_DIM, B), jnp.float32),
        mesh=mesh,
        compiler_params=pltpu.CompilerParams(needs_layout_passes=False),
        scratch_types=[
            pltpu.VMEM((VOCAB,), jnp.float32),
            pltpu.VMEM((BH,), jnp.int32),
            pltpu.VMEM((BH,), jnp.float32),
        ],
    )
    def gather_kernel(tabT_hbm, idx_hbm, out_hbm, tab_v, idx_v, out_v):
        e = lax.axis_index("subcore") * 2 + lax.axis_index("core")

        @pl.loop(0, NTAB)
        def _(t):
            pltpu.sync_copy(tabT_hbm.at[t, e], tab_v)

            @pl.loop(0, B // BH)
            def _(c):
                pltpu.sync_copy(idx_hbm.at[t, pl.ds(c * BH, BH)], idx_v)

                @pl.loop(0, BH // 16)
                def _(i):
                    iv = idx_v[pl.ds(i * 16, 16)]
                    out_v[pl.ds(i * 16, 16)] = plsc.load_gather(tab_v, [iv])

                pltpu.sync_copy(out_v, out_hbm.at[t * EDIM + e, pl.ds(c * BH, BH)])

    return gather_kernel(tabT, idx)


def _mlp_t_body(xT_ref, numT_ref, w1cT_ref, w1nT_ref, b1_ref, w2T_ref, b2_ref,
                w3T_ref, b3_ref, w4T_ref, b4_ref, outT_ref):
    h = jnp.dot(w1cT_ref[...], xT_ref[...], preferred_element_type=jnp.float32)
    h = h + jnp.dot(w1nT_ref[...], numT_ref[...], preferred_element_type=jnp.float32)
    h = jnp.maximum(h + b1_ref[...], 0.0)
    h = jnp.maximum(
        jnp.dot(w2T_ref[...], h, preferred_element_type=jnp.float32) + b2_ref[...], 0.0)
    h = jnp.maximum(
        jnp.dot(w3T_ref[...], h, preferred_element_type=jnp.float32) + b3_ref[...], 0.0)
    outT_ref[...] = jnp.dot(w4T_ref[...], h, preferred_element_type=jnp.float32) + b4_ref[...]


def _mlp_t(xT, numT, w1cT, w1nT, b1c, w2T, b2c, w3T, b3c, w4T, b4c):
    nblk = B // BN
    full = lambda shape: pl.BlockSpec(shape, lambda i: (0, 0))
    return pl.pallas_call(
        _mlp_t_body,
        grid=(nblk,),
        in_specs=[
            pl.BlockSpec((CAT_DIM, BN), lambda i: (0, i)),
            pl.BlockSpec((NUM_DENSE, BN), lambda i: (0, i)),
            full((512, CAT_DIM)),
            full((512, NUM_DENSE)),
            full((512, 1)),
            full((256, 512)),
            full((256, 1)),
            full((128, 256)),
            full((128, 1)),
            full((1, 128)),
            full((1, 1)),
        ],
        out_specs=pl.BlockSpec((1, BN), lambda i: (0, i)),
        out_shape=jax.ShapeDtypeStruct((1, B), jnp.float32),
    )(xT, numT, w1cT, w1nT, b1c, w2T, b2c, w3T, b3c, w4T, b4c)


def kernel(numerical_features, cat_features, tables, W1, b1, W2, b2, W3, b3, W4, b4):
    idx = jnp.mod(cat_features[:, :, 0], VOCAB)
    tabT = jnp.transpose(tables, (0, 2, 1))
    xT = _sc_gather_t(tabT, idx)
    numT = numerical_features.T
    outT = _mlp_t(
        xT,
        numT,
        W1[:CAT_DIM].T,
        W1[CAT_DIM:].T,
        b1.reshape(-1, 1),
        W2.T,
        b2.reshape(-1, 1),
        W3.T,
        b3.reshape(-1, 1),
        W4.T,
        b4.reshape(-1, 1),
    )
    return outT.reshape(B, 1)


# speedup vs baseline: 36.3865x; 1.4334x over previous
import functools

import jax
import jax.numpy as jnp
from jax import lax
from jax.experimental import pallas as pl
from jax.experimental.pallas import tpu as pltpu
from jax.experimental.pallas import tpu_sc as plsc

VOCAB = 100000
EDIM = 32
NTAB = 26
B = 16384
NUM_DENSE = 13
CAT_DIM = NTAB * EDIM

BH = B // 2
BN = 2048


def _sc_gather_t(tabT, idx):
    mesh = plsc.VectorSubcoreMesh(core_axis_name="core", subcore_axis_name="subcore")

    @functools.partial(
        pl.kernel,
        out_type=jax.ShapeDtypeStruct((CAT_DIM, B), jnp.float32),
        mesh=mesh,
        compiler_params=pltpu.CompilerParams(needs_layout_passes=False),
        scratch_types=[
            pltpu.VMEM((VOCAB,), jnp.float32),
            pltpu.VMEM((BH,), jnp.int32),
            pltpu.VMEM((BH,), jnp.float32),
        ],
    )
    def gather_kernel(tabT_hbm, idx_hbm, out_hbm, tab_v, idx_v, out_v):
        e = lax.axis_index("subcore") * 2 + lax.axis_index("core")
        t0 = (e * NTAB) // 32

        @pl.loop(0, NTAB)
        def _(k):
            t = lax.rem(t0 + k, NTAB)
            pltpu.sync_copy(tabT_hbm.at[t, e], tab_v)

            @pl.loop(0, B // BH)
            def _(c):
                pltpu.sync_copy(idx_hbm.at[t, pl.ds(c * BH, BH)], idx_v)

                @plsc.parallel_loop(0, BH // 16, unroll=8)
                def _(i):
                    iv = idx_v[pl.ds(i * 16, 16)]
                    out_v[pl.ds(i * 16, 16)] = plsc.load_gather(tab_v, [iv])

                pltpu.sync_copy(out_v, out_hbm.at[t * EDIM + e, pl.ds(c * BH, BH)])

    return gather_kernel(tabT, idx)


def _mlp_t_body(xT_ref, numT_ref, w1cT_ref, w1nT_ref, b1_ref, w2T_ref, b2_ref,
                w3T_ref, b3_ref, w4T_ref, b4_ref, outT_ref):
    h = jnp.dot(w1cT_ref[...], xT_ref[...], preferred_element_type=jnp.float32)
    h = h + jnp.dot(w1nT_ref[...], numT_ref[...], preferred_element_type=jnp.float32)
    h = jnp.maximum(h + b1_ref[...], 0.0)
    h = jnp.maximum(
        jnp.dot(w2T_ref[...], h, preferred_element_type=jnp.float32) + b2_ref[...], 0.0)
    h = jnp.maximum(
        jnp.dot(w3T_ref[...], h, preferred_element_type=jnp.float32) + b3_ref[...], 0.0)
    outT_ref[...] = jnp.dot(w4T_ref[...], h, preferred_element_type=jnp.float32) + b4_ref[...]


def _mlp_t(xT, numT, w1cT, w1nT, b1c, w2T, b2c, w3T, b3c, w4T, b4c):
    nblk = B // BN
    full = lambda shape: pl.BlockSpec(shape, lambda i: (0, 0))
    return pl.pallas_call(
        _mlp_t_body,
        grid=(nblk,),
        in_specs=[
            pl.BlockSpec((CAT_DIM, BN), lambda i: (0, i)),
            pl.BlockSpec((NUM_DENSE, BN), lambda i: (0, i)),
            full((512, CAT_DIM)),
            full((512, NUM_DENSE)),
            full((512, 1)),
            full((256, 512)),
            full((256, 1)),
            full((128, 256)),
            full((128, 1)),
            full((1, 128)),
            full((1, 1)),
        ],
        out_specs=pl.BlockSpec((1, BN), lambda i: (0, i)),
        out_shape=jax.ShapeDtypeStruct((1, B), jnp.float32),
    )(xT, numT, w1cT, w1nT, b1c, w2T, b2c, w3T, b3c, w4T, b4c)


def kernel(numerical_features, cat_features, tables, W1, b1, W2, b2, W3, b3, W4, b4):
    idx = jnp.mod(cat_features[:, :, 0], VOCAB)
    tabT = jnp.transpose(tables, (0, 2, 1))
    xT = _sc_gather_t(tabT, idx)
    numT = numerical_features.T
    outT = _mlp_t(
        xT,
        numT,
        W1[:CAT_DIM].T,
        W1[CAT_DIM:].T,
        b1.reshape(-1, 1),
        W2.T,
        b2.reshape(-1, 1),
        W3.T,
        b3.reshape(-1, 1),
        W4.T,
        b4.reshape(-1, 1),
    )
    return outT.reshape(B, 1)


# speedup vs baseline: 40.9565x; 1.1256x over previous
import functools

import jax
import jax.numpy as jnp
from jax import lax
from jax.experimental import pallas as pl
from jax.experimental.pallas import tpu as pltpu
from jax.experimental.pallas import tpu_sc as plsc

VOCAB = 100000
EDIM = 32
NTAB = 26
B = 16384
NUM_DENSE = 13
CAT_DIM = NTAB * EDIM

BC = 4096
NCHUNK = B // BC
BN = 2048


def _sc_gather_t(tabT, idx):
    mesh = plsc.VectorSubcoreMesh(core_axis_name="core", subcore_axis_name="subcore")

    @functools.partial(
        pl.kernel,
        out_type=jax.ShapeDtypeStruct((CAT_DIM, B), jnp.float32),
        mesh=mesh,
        compiler_params=pltpu.CompilerParams(needs_layout_passes=False),
        scratch_types=[
            pltpu.VMEM((VOCAB,), jnp.float32),
            pltpu.VMEM((BC,), jnp.int32),
            pltpu.VMEM((BC,), jnp.int32),
            pltpu.VMEM((BC,), jnp.float32),
            pltpu.VMEM((BC,), jnp.float32),
            pltpu.SemaphoreType.DMA,
            pltpu.SemaphoreType.DMA,
            pltpu.SemaphoreType.DMA,
            pltpu.SemaphoreType.DMA,
        ],
    )
    def gather_kernel(tabT_hbm, idx_hbm, out_hbm, tab_v, idx_va, idx_vb,
                      out_va, out_vb, sem_i0, sem_i1, sem_o0, sem_o1):
        e = lax.axis_index("subcore") * 2 + lax.axis_index("core")
        t0 = (e * NTAB) // 32
        sem_i = (sem_i0, sem_i1)
        sem_o = (sem_o0, sem_o1)
        idx_bufs = (idx_va, idx_vb)
        out_bufs = (out_va, out_vb)

        def start_idx(t, c, buf):
            pltpu.async_copy(
                idx_hbm.at[t, pl.ds(c * BC, BC)], idx_bufs[buf], sem_i[buf])

        def wait_idx(buf):
            pltpu.make_async_copy(
                idx_hbm.at[0, pl.ds(0, BC)], idx_bufs[buf], sem_i[buf]).wait()

        def wait_out(buf):
            pltpu.make_async_copy(
                out_bufs[buf], out_hbm.at[0, pl.ds(0, BC)], sem_o[buf]).wait()

        start_idx(t0, 0, 0)

        @pl.loop(0, NTAB)
        def _(k):
            t = lax.rem(t0 + k, NTAB)
            t_next = lax.rem(t0 + k + 1, NTAB)
            pltpu.sync_copy(tabT_hbm.at[t, e], tab_v)

            for c in range(NCHUNK):
                buf = c % 2
                wait_idx(buf)
                if c < NCHUNK - 1:
                    start_idx(t, c + 1, (c + 1) % 2)
                else:
                    @pl.when(k < NTAB - 1)
                    def _():
                        start_idx(t_next, 0, 0)
                if c >= 2:
                    wait_out(buf)
                else:
                    @pl.when(k > 0)
                    def _():
                        wait_out(buf)

                ib, ob = idx_bufs[buf], out_bufs[buf]

                @plsc.parallel_loop(0, BC // 16, unroll=8)
                def _(i):
                    iv = ib[pl.ds(i * 16, 16)]
                    ob[pl.ds(i * 16, 16)] = plsc.load_gather(tab_v, [iv])

                pltpu.async_copy(
                    ob, out_hbm.at[t * EDIM + e, pl.ds(c * BC, BC)], sem_o[buf])

        wait_out(0)
        wait_out(1)

    return gather_kernel(tabT, idx)


def _mlp_t_body(xT_ref, numT_ref, w1cT_ref, w1nT_ref, b1_ref, w2T_ref, b2_ref,
                w3T_ref, b3_ref, w4T_ref, b4_ref, outT_ref):
    h = jnp.dot(w1cT_ref[...], xT_ref[...], preferred_element_type=jnp.float32)
    h = h + jnp.dot(w1nT_ref[...], numT_ref[...], preferred_element_type=jnp.float32)
    h = jnp.maximum(h + b1_ref[...], 0.0)
    h = jnp.maximum(
        jnp.dot(w2T_ref[...], h, preferred_element_type=jnp.float32) + b2_ref[...], 0.0)
    h = jnp.maximum(
        jnp.dot(w3T_ref[...], h, preferred_element_type=jnp.float32) + b3_ref[...], 0.0)
    outT_ref[...] = jnp.dot(w4T_ref[...], h, preferred_element_type=jnp.float32) + b4_ref[...]


def _mlp_t(xT, numT, w1cT, w1nT, b1c, w2T, b2c, w3T, b3c, w4T, b4c):
    nblk = B // BN
    full = lambda shape: pl.BlockSpec(shape, lambda i: (0, 0))
    return pl.pallas_call(
        _mlp_t_body,
        grid=(nblk,),
        in_specs=[
            pl.BlockSpec((CAT_DIM, BN), lambda i: (0, i)),
            pl.BlockSpec((NUM_DENSE, BN), lambda i: (0, i)),
            full((512, CAT_DIM)),
            full((512, NUM_DENSE)),
            full((512, 1)),
            full((256, 512)),
            full((256, 1)),
            full((128, 256)),
            full((128, 1)),
            full((1, 128)),
            full((1, 1)),
        ],
        out_specs=pl.BlockSpec((1, BN), lambda i: (0, i)),
        out_shape=jax.ShapeDtypeStruct((1, B), jnp.float32),
    )(xT, numT, w1cT, w1nT, b1c, w2T, b2c, w3T, b3c, w4T, b4c)


def kernel(numerical_features, cat_features, tables, W1, b1, W2, b2, W3, b3, W4, b4):
    idx = jnp.mod(cat_features[:, :, 0], VOCAB)
    tabT = jnp.transpose(tables, (0, 2, 1))
    xT = _sc_gather_t(tabT, idx)
    numT = numerical_features.T
    outT = _mlp_t(
        xT,
        numT,
        W1[:CAT_DIM].T,
        W1[CAT_DIM:].T,
        b1.reshape(-1, 1),
        W2.T,
        b2.reshape(-1, 1),
        W3.T,
        b3.reshape(-1, 1),
        W4.T,
        b4.reshape(-1, 1),
    )
    return outT.reshape(B, 1)
